# baseline (speedup 1.0000x reference)
import functools

import jax
import jax.numpy as jnp
from jax import lax
from jax.experimental import pallas as pl
from jax.experimental.pallas import tpu as pltpu
from jax.experimental.pallas import tpu_sc as plsc

NPAD = 1_048_576
NW = 32
PER_W = NPAD // NW
CHUNK = 2048
NCHUNK = PER_W // CHUNK
LANES = 16
VECS = CHUNK // LANES
GRID = 4096
NCELL = (GRID - 2) * GRID + GRID - 2 + 2
QROWS = NCELL // 2


def _body(pts_hbm, zq_hbm, out_hbm,
          pb0, pb1, ib0, ib1, vq0, vq1, ob0, ob1, sem0, sem1):
    pb = (pb0, pb1)
    ib = (ib0, ib1)
    vq = (vq0, vq1)
    ob = (ob0, ob1)
    sem = (sem0, sem1)

    c = lax.axis_index("c")
    s = lax.axis_index("s")
    base = (s * 2 + c) * PER_W

    lane = lax.iota(jnp.int32, LANES)
    lane2 = lane * 2

    def load_pts(p, off):
        pltpu.sync_copy(pts_hbm.at[pl.ds(off * 2, CHUNK * 2)], pb[p])

    def point_xy(p, i):
        xv = plsc.load_gather(pb[p], [i * (2 * LANES) + lane2])
        yv = plsc.load_gather(pb[p], [i * (2 * LANES) + lane2 + 1])
        return xv, yv

    def cell_xy(xv, yv):
        ix = jnp.clip(xv.astype(jnp.int32), 0, GRID - 2)
        iy = jnp.clip(yv.astype(jnp.int32), 0, GRID - 2)
        return ix, iy

    def compute_idx(p):
        def body(i, carry):
            xv, yv = point_xy(p, i)
            ix, iy = cell_xy(xv, yv)
            cell = ix * GRID + iy
            ib[p][pl.ds(i * LANES, LANES)] = lax.shift_right_logical(cell, 1)
            return carry

        lax.fori_loop(0, VECS, body, 0)

    def fire(p):
        return pltpu.async_copy(zq_hbm.at[ib[p]], vq[p], sem[p])

    def mix(p, off):
        def body(i, carry):
            xv, yv = point_xy(p, i)
            ix, iy = cell_xy(xv, yv)
            wx = xv - ix.astype(jnp.float32)
            wy = yv - iy.astype(jnp.float32)
            row = i * LANES + lane
            cb = (iy & 1) * 4
            z00 = plsc.load_gather(vq[p], [row, cb])
            z01 = plsc.load_gather(vq[p], [row, cb + 1])
            z10 = plsc.load_gather(vq[p], [row, cb + 2])
            z11 = plsc.load_gather(vq[p], [row, cb + 3])
            a = z00 + (z01 - z00) * wy
            b = z10 + (z11 - z10) * wy
            ob[p][pl.ds(i * LANES, LANES)] = a + (b - a) * wx
            return carry

        lax.fori_loop(0, VECS, body, 0)
        pltpu.sync_copy(ob[p], out_hbm.at[pl.ds(off, CHUNK)])

    load_pts(0, base)
    compute_idx(0)
    cp = fire(0)
    for ci in range(NCHUNK):
        p = ci & 1
        q = p ^ 1
        nxt = None
        if ci + 1 < NCHUNK:
            load_pts(q, base + (ci + 1) * CHUNK)
            compute_idx(q)
            nxt = fire(q)
        cp.wait()
        mix(p, base + ci * CHUNK)
        cp = nxt


_interp = functools.partial(
    pl.kernel,
    out_type=jax.ShapeDtypeStruct((NPAD,), jnp.float32),
    mesh=plsc.VectorSubcoreMesh(core_axis_name="c", subcore_axis_name="s"),
    compiler_params=pltpu.CompilerParams(
        needs_layout_passes=False, use_tc_tiling_on_sc=False),
    scratch_types=[
        pltpu.VMEM((CHUNK * 2,), jnp.float32),
        pltpu.VMEM((CHUNK * 2,), jnp.float32),
        pltpu.VMEM((CHUNK,), jnp.int32),
        pltpu.VMEM((CHUNK,), jnp.int32),
        pltpu.VMEM((CHUNK, 8), jnp.float32),
        pltpu.VMEM((CHUNK, 8), jnp.float32),
        pltpu.VMEM((CHUNK,), jnp.float32),
        pltpu.VMEM((CHUNK,), jnp.float32),
        pltpu.SemaphoreType.DMA,
        pltpu.SemaphoreType.DMA,
    ],
)(_body)


def kernel(points_to_interpolate, xs, ys, zs, repeats=1):
    n = points_to_interpolate.shape[0]
    pts = jnp.pad(points_to_interpolate, ((0, NPAD - n), (0, 0)))
    zf = jnp.concatenate([zs.reshape(-1), jnp.zeros((1,), jnp.float32)])
    one = jnp.minimum(jnp.abs(zs[0, 0]) + 1.0, 1.0)
    zq = jnp.stack(
        [zf[o:o + NCELL] for o in (0, 1, GRID, GRID + 1)], axis=1) * one
    out = _interp(pts.reshape(-1), zq.reshape(QROWS, 8))
    return out[:n]


# speedup vs baseline: 1.5505x; 1.5505x over previous
import functools

import jax
import jax.numpy as jnp
from jax import lax
from jax.experimental import pallas as pl
from jax.experimental.pallas import tpu as pltpu
from jax.experimental.pallas import tpu_sc as plsc

NPAD = 1_048_576
NW = 32
PER_W = NPAD // NW
CHUNK = 2048
NCHUNK = PER_W // CHUNK
LANES = 16
VECS = CHUNK // LANES
GRID = 4096
NCELL = (GRID - 2) * GRID + GRID - 2 + 2
QROWS = NCELL // 2


def _body(x_hbm, y_hbm, zq_hbm, out_hbm,
          xb0, xb1, yb0, yb1, ib0, ib1, vq0, vq1, ob0, ob1, sem0, sem1):
    xb = (xb0, xb1)
    yb = (yb0, yb1)
    ib = (ib0, ib1)
    vq = (vq0, vq1)
    ob = (ob0, ob1)
    sem = (sem0, sem1)

    c = lax.axis_index("c")
    s = lax.axis_index("s")
    base = (s * 2 + c) * PER_W

    lane = lax.iota(jnp.int32, LANES)

    def load_xy(p, off):
        pltpu.sync_copy(x_hbm.at[pl.ds(off, CHUNK)], xb[p])
        pltpu.sync_copy(y_hbm.at[pl.ds(off, CHUNK)], yb[p])

    def cell_xy(xv, yv):
        ix = jnp.clip(xv.astype(jnp.int32), 0, GRID - 2)
        iy = jnp.clip(yv.astype(jnp.int32), 0, GRID - 2)
        return ix, iy

    def compute_idx(p):
        def body(i, carry):
            sl = pl.ds(i * LANES, LANES)
            ix, iy = cell_xy(xb[p][sl], yb[p][sl])
            cell = ix * GRID + iy
            ib[p][sl] = lax.shift_right_logical(cell, 1)
            return carry

        lax.fori_loop(0, VECS, body, 0)

    def fire(p):
        return pltpu.async_copy(zq_hbm.at[ib[p]], vq[p], sem[p])

    def mix(p, off):
        def body(i, carry):
            sl = pl.ds(i * LANES, LANES)
            xv = xb[p][sl]
            yv = yb[p][sl]
            ix, iy = cell_xy(xv, yv)
            wx = xv - ix.astype(jnp.float32)
            wy = yv - iy.astype(jnp.float32)
            row = i * LANES + lane
            cb = (iy & 1) * 4
            z00 = plsc.load_gather(vq[p], [row, cb])
            z01 = plsc.load_gather(vq[p], [row, cb + 1])
            z10 = plsc.load_gather(vq[p], [row, cb + 2])
            z11 = plsc.load_gather(vq[p], [row, cb + 3])
            a = z00 + (z01 - z00) * wy
            b = z10 + (z11 - z10) * wy
            ob[p][sl] = a + (b - a) * wx
            return carry

        lax.fori_loop(0, VECS, body, 0)
        pltpu.sync_copy(ob[p], out_hbm.at[pl.ds(off, CHUNK)])

    load_xy(0, base)
    compute_idx(0)
    cp = fire(0)
    for ci in range(NCHUNK):
        p = ci & 1
        q = p ^ 1
        nxt = None
        if ci + 1 < NCHUNK:
            load_xy(q, base + (ci + 1) * CHUNK)
            compute_idx(q)
            nxt = fire(q)
        cp.wait()
        mix(p, base + ci * CHUNK)
        cp = nxt


_interp = functools.partial(
    pl.kernel,
    out_type=jax.ShapeDtypeStruct((NPAD,), jnp.float32),
    mesh=plsc.VectorSubcoreMesh(core_axis_name="c", subcore_axis_name="s"),
    compiler_params=pltpu.CompilerParams(
        needs_layout_passes=False, use_tc_tiling_on_sc=False),
    scratch_types=[
        pltpu.VMEM((CHUNK,), jnp.float32),
        pltpu.VMEM((CHUNK,), jnp.float32),
        pltpu.VMEM((CHUNK,), jnp.float32),
        pltpu.VMEM((CHUNK,), jnp.float32),
        pltpu.VMEM((CHUNK,), jnp.int32),
        pltpu.VMEM((CHUNK,), jnp.int32),
        pltpu.VMEM((CHUNK, 8), jnp.float32),
        pltpu.VMEM((CHUNK, 8), jnp.float32),
        pltpu.VMEM((CHUNK,), jnp.float32),
        pltpu.VMEM((CHUNK,), jnp.float32),
        pltpu.SemaphoreType.DMA,
        pltpu.SemaphoreType.DMA,
    ],
)(_body)


def kernel(points_to_interpolate, xs, ys, zs, repeats=1):
    n = points_to_interpolate.shape[0]
    xp = jnp.pad(points_to_interpolate[:, 0], (0, NPAD - n))
    yp = jnp.pad(points_to_interpolate[:, 1], (0, NPAD - n))
    zf = jnp.concatenate([zs.reshape(-1), jnp.zeros((2,), jnp.float32)])
    one = jnp.minimum(jnp.abs(zs[0, 0]) + 1.0, 1.0)
    zq = jnp.stack(
        [zf[o:o + 2 * QROWS:2]
         for o in (0, 1, GRID, GRID + 1, 1, 2, GRID + 1, GRID + 2)],
        axis=1) * one
    out = _interp(xp, yp, zq)
    return out[:n]


# speedup vs baseline: 33.0296x; 21.3021x over previous
import functools

import jax
import jax.numpy as jnp
from jax import lax
from jax.experimental import pallas as pl
from jax.experimental.pallas import tpu as pltpu
from jax.experimental.pallas import tpu_sc as plsc

NPAD = 1_048_576
NW = 32
PER_W = NPAD // NW
CHUNK = 2048
NCHUNK = PER_W // CHUNK
LANES = 16
VECS = CHUNK // LANES
GRID = 4096
NCELL = (GRID - 2) * GRID + GRID - 2 + 2
ROWS16 = NCELL // 4

BCELL = NCELL // NW
BCH = 8192
BSIZES = [BCH] * 63 + [BCELL - 63 * BCH]


def _build_body(zf_hbm, tab_hbm, slab1, slab2, ob0, ob1, sem0, sem1):
    ob = (ob0, ob1)
    sem = (sem0, sem1)
    c = lax.axis_index("c")
    s = lax.axis_index("s")
    base = (s * 2 + c) * BCELL

    lane = lax.iota(jnp.int32, LANES)
    rc = lax.shift_right_logical(lane, 2)
    cv = (lane & 3) * 4

    cps = [None, None]
    off = 0
    for ci, cb in enumerate(BSIZES):
        c0 = base + off
        last = ci == len(BSIZES) - 1
        b2 = 16 if last else 0
        pltpu.sync_copy(zf_hbm.at[pl.ds(c0, cb + 16)], slab1.at[pl.ds(0, cb + 16)])
        if last:
            is_last_w = (s * 2 + c) == NW - 1

            @pl.when(is_last_w)
            def _():
                pltpu.sync_copy(zf_hbm.at[pl.ds(c0 + GRID - b2, cb + 16)],
                                slab2.at[pl.ds(0, cb + 16)])

            @pl.when(jnp.logical_not(is_last_w))
            def _():
                pltpu.sync_copy(zf_hbm.at[pl.ds(c0 + GRID - b2, cb + 24)],
                                slab2.at[pl.ds(0, cb + 24)])
        else:
            pltpu.sync_copy(zf_hbm.at[pl.ds(c0 + GRID - b2, cb + 16)],
                            slab2.at[pl.ds(0, cb + 16)])
        p = ci & 1
        if cps[p] is not None:
            cps[p].wait()

        def it(i, carry, p=p, b2=b2):
            for g in range(2):
                u0 = i * 32 + g * 16
                z00 = slab1[pl.ds(u0, LANES)]
                z01 = slab1[pl.ds(u0 + 1, LANES)]
                z10 = slab2[pl.ds(u0 + b2, LANES)]
                z11 = slab2[pl.ds(u0 + b2 + 1, LANES)]
                rows = i * 8 + (g * 4) + rc
                plsc.store_scatter(ob[p], [rows, cv], z00)
                plsc.store_scatter(ob[p], [rows, cv + 1], z01)
                plsc.store_scatter(ob[p], [rows, cv + 2], z10)
                plsc.store_scatter(ob[p], [rows, cv + 3], z11)
            return carry

        lax.fori_loop(0, cb // 32, it, 0)
        cps[p] = pltpu.async_copy(
            ob[p].at[pl.ds(0, cb // 4)], tab_hbm.at[pl.ds(c0 // 4, cb // 4)],
            sem[p])
        off += cb
    for cp in cps:
        if cp is not None:
            cp.wait()


_build = functools.partial(
    pl.kernel,
    out_type=jax.ShapeDtypeStruct((ROWS16, 16), jnp.float32),
    mesh=plsc.VectorSubcoreMesh(core_axis_name="c", subcore_axis_name="s"),
    compiler_params=pltpu.CompilerParams(
        needs_layout_passes=False, use_tc_tiling_on_sc=False),
    scratch_types=[
        pltpu.VMEM((BCH + 16,), jnp.float32),
        pltpu.VMEM((BCH + 16,), jnp.float32),
        pltpu.VMEM((BCH // 4, 16), jnp.float32),
        pltpu.VMEM((BCH // 4, 16), jnp.float32),
        pltpu.SemaphoreType.DMA,
        pltpu.SemaphoreType.DMA,
    ],
)(_build_body)


def _interp_body(x_hbm, y_hbm, zq_hbm, out_hbm,
                 xb0, xb1, yb0, yb1, ib0, ib1, vq0, vq1, ob0, ob1,
                 sem0, sem1):
    xb = (xb0, xb1)
    yb = (yb0, yb1)
    ib = (ib0, ib1)
    vq = (vq0, vq1)
    ob = (ob0, ob1)
    sem = (sem0, sem1)

    c = lax.axis_index("c")
    s = lax.axis_index("s")
    base = (s * 2 + c) * PER_W

    lane = lax.iota(jnp.int32, LANES)

    def load_xy(p, off):
        pltpu.sync_copy(x_hbm.at[pl.ds(off, CHUNK)], xb[p])
        pltpu.sync_copy(y_hbm.at[pl.ds(off, CHUNK)], yb[p])

    def cell_xy(xv, yv):
        ix = jnp.clip(xv.astype(jnp.int32), 0, GRID - 2)
        iy = jnp.clip(yv.astype(jnp.int32), 0, GRID - 2)
        return ix, iy

    def compute_idx(p):
        def body(i, carry):
            sl = pl.ds(i * LANES, LANES)
            ix, iy = cell_xy(xb[p][sl], yb[p][sl])
            cell = ix * GRID + iy
            ib[p][sl] = lax.shift_right_logical(cell, 2)
            return carry

        lax.fori_loop(0, VECS, body, 0)

    def fire(p):
        return pltpu.async_copy(zq_hbm.at[ib[p]], vq[p], sem[p])

    def mix(p, off):
        def body(i, carry):
            sl = pl.ds(i * LANES, LANES)
            xv = xb[p][sl]
            yv = yb[p][sl]
            ix, iy = cell_xy(xv, yv)
            wx = xv - ix.astype(jnp.float32)
            wy = yv - iy.astype(jnp.float32)
            row = i * LANES + lane
            cb = ((ix * GRID + iy) & 3) * 4
            z00 = plsc.load_gather(vq[p], [row, cb])
            z01 = plsc.load_gather(vq[p], [row, cb + 1])
            z10 = plsc.load_gather(vq[p], [row, cb + 2])
            z11 = plsc.load_gather(vq[p], [row, cb + 3])
            a = z00 + (z01 - z00) * wy
            b = z10 + (z11 - z10) * wy
            ob[p][sl] = a + (b - a) * wx
            return carry

        lax.fori_loop(0, VECS, body, 0)
        pltpu.sync_copy(ob[p], out_hbm.at[pl.ds(off, CHUNK)])

    load_xy(0, base)
    compute_idx(0)
    cp = fire(0)
    for ci in range(NCHUNK):
        p = ci & 1
        q = p ^ 1
        nxt = None
        if ci + 1 < NCHUNK:
            load_xy(q, base + (ci + 1) * CHUNK)
            compute_idx(q)
            nxt = fire(q)
        cp.wait()
        mix(p, base + ci * CHUNK)
        cp = nxt


_interp = functools.partial(
    pl.kernel,
    out_type=jax.ShapeDtypeStruct((NPAD,), jnp.float32),
    mesh=plsc.VectorSubcoreMesh(core_axis_name="c", subcore_axis_name="s"),
    compiler_params=pltpu.CompilerParams(
        needs_layout_passes=False, use_tc_tiling_on_sc=False),
    scratch_types=[
        pltpu.VMEM((CHUNK,), jnp.float32),
        pltpu.VMEM((CHUNK,), jnp.float32),
        pltpu.VMEM((CHUNK,), jnp.float32),
        pltpu.VMEM((CHUNK,), jnp.float32),
        pltpu.VMEM((CHUNK,), jnp.int32),
        pltpu.VMEM((CHUNK,), jnp.int32),
        pltpu.VMEM((CHUNK, 16), jnp.float32),
        pltpu.VMEM((CHUNK, 16), jnp.float32),
        pltpu.VMEM((CHUNK,), jnp.float32),
        pltpu.VMEM((CHUNK,), jnp.float32),
        pltpu.SemaphoreType.DMA,
        pltpu.SemaphoreType.DMA,
    ],
)(_interp_body)


def kernel(points_to_interpolate, xs, ys, zs, repeats=1):
    n = points_to_interpolate.shape[0]
    xp = jnp.pad(points_to_interpolate[:, 0], (0, NPAD - n))
    yp = jnp.pad(points_to_interpolate[:, 1], (0, NPAD - n))
    table = _build(zs.reshape(-1))
    out = _interp(xp, yp, table)
    return out[:n]


# speedup vs baseline: 33.1720x; 1.0043x over previous
import functools

import jax
import jax.numpy as jnp
from jax import lax
from jax.experimental import pallas as pl
from jax.experimental.pallas import tpu as pltpu
from jax.experimental.pallas import tpu_sc as plsc

NPAD = 1_048_576
NW = 32
PER_W = NPAD // NW
CHUNK = 2048
NCHUNK = PER_W // CHUNK
LANES = 16
VECS = CHUNK // LANES
GRID = 4096
NCELL = (GRID - 2) * GRID + GRID - 2 + 2
ROWS16 = NCELL // 4

BCELL = NCELL // NW
BCH = 8192
BSIZES = [BCH] * 63 + [BCELL - 63 * BCH]


def _build_body(zf_hbm, tab_hbm, slab1, slab2, ob0, ob1, sem0, sem1):
    ob = (ob0, ob1)
    sem = (sem0, sem1)
    c = lax.axis_index("c")
    s = lax.axis_index("s")
    base = (s * 2 + c) * BCELL

    lane = lax.iota(jnp.int32, LANES)
    rc = lax.shift_right_logical(lane, 2)
    cv = (lane & 3) * 4

    cps = [None, None]
    off = 0
    for ci, cb in enumerate(BSIZES):
        c0 = base + off
        last = ci == len(BSIZES) - 1
        b2 = 16 if last else 0
        pltpu.sync_copy(zf_hbm.at[pl.ds(c0, cb + 16)], slab1.at[pl.ds(0, cb + 16)])
        if last:
            is_last_w = (s * 2 + c) == NW - 1

            @pl.when(is_last_w)
            def _():
                pltpu.sync_copy(zf_hbm.at[pl.ds(c0 + GRID - b2, cb + 16)],
                                slab2.at[pl.ds(0, cb + 16)])

            @pl.when(jnp.logical_not(is_last_w))
            def _():
                pltpu.sync_copy(zf_hbm.at[pl.ds(c0 + GRID - b2, cb + 24)],
                                slab2.at[pl.ds(0, cb + 24)])
        else:
            pltpu.sync_copy(zf_hbm.at[pl.ds(c0 + GRID - b2, cb + 16)],
                            slab2.at[pl.ds(0, cb + 16)])
        p = ci & 1
        if cps[p] is not None:
            cps[p].wait()

        def it(i, carry, p=p, b2=b2):
            for g in range(4):
                u0 = i * 64 + g * 16
                z00 = slab1[pl.ds(u0, LANES)]
                z01 = slab1[pl.ds(u0 + 1, LANES)]
                z10 = slab2[pl.ds(u0 + b2, LANES)]
                z11 = slab2[pl.ds(u0 + b2 + 1, LANES)]
                rows = i * 16 + (g * 4) + rc
                plsc.store_scatter(ob[p], [rows, cv], z00)
                plsc.store_scatter(ob[p], [rows, cv + 1], z01)
                plsc.store_scatter(ob[p], [rows, cv + 2], z10)
                plsc.store_scatter(ob[p], [rows, cv + 3], z11)
            return carry

        lax.fori_loop(0, cb // 64, it, 0)
        cps[p] = pltpu.async_copy(
            ob[p].at[pl.ds(0, cb // 4)], tab_hbm.at[pl.ds(c0 // 4, cb // 4)],
            sem[p])
        off += cb
    for cp in cps:
        if cp is not None:
            cp.wait()


_build = functools.partial(
    pl.kernel,
    out_type=jax.ShapeDtypeStruct((ROWS16, 16), jnp.float32),
    mesh=plsc.VectorSubcoreMesh(core_axis_name="c", subcore_axis_name="s"),
    compiler_params=pltpu.CompilerParams(
        needs_layout_passes=False, use_tc_tiling_on_sc=False),
    scratch_types=[
        pltpu.VMEM((BCH + 16,), jnp.float32),
        pltpu.VMEM((BCH + 16,), jnp.float32),
        pltpu.VMEM((BCH // 4, 16), jnp.float32),
        pltpu.VMEM((BCH // 4, 16), jnp.float32),
        pltpu.SemaphoreType.DMA,
        pltpu.SemaphoreType.DMA,
    ],
)(_build_body)


def _interp_body(x_hbm, y_hbm, zq_hbm, out_hbm,
                 xb0, xb1, yb0, yb1, ib0, ib1, vq0, vq1, ob0, ob1,
                 sem0, sem1):
    xb = (xb0, xb1)
    yb = (yb0, yb1)
    ib = (ib0, ib1)
    vq = (vq0, vq1)
    ob = (ob0, ob1)
    sem = (sem0, sem1)

    c = lax.axis_index("c")
    s = lax.axis_index("s")
    base = (s * 2 + c) * PER_W

    lane = lax.iota(jnp.int32, LANES)

    def load_xy(p, off):
        pltpu.sync_copy(x_hbm.at[pl.ds(off, CHUNK)], xb[p])
        pltpu.sync_copy(y_hbm.at[pl.ds(off, CHUNK)], yb[p])

    def cell_xy(xv, yv):
        ix = jnp.clip(xv.astype(jnp.int32), 0, GRID - 2)
        iy = jnp.clip(yv.astype(jnp.int32), 0, GRID - 2)
        return ix, iy

    def compute_idx(p):
        def body(i, carry):
            sl = pl.ds(i * LANES, LANES)
            ix, iy = cell_xy(xb[p][sl], yb[p][sl])
            cell = ix * GRID + iy
            ib[p][sl] = lax.shift_right_logical(cell, 2)
            return carry

        lax.fori_loop(0, VECS, body, 0)

    def fire(p):
        return pltpu.async_copy(zq_hbm.at[ib[p]], vq[p], sem[p])

    def mix(p, off):
        def body(i, carry):
            sl = pl.ds(i * LANES, LANES)
            xv = xb[p][sl]
            yv = yb[p][sl]
            ix, iy = cell_xy(xv, yv)
            wx = xv - ix.astype(jnp.float32)
            wy = yv - iy.astype(jnp.float32)
            row = i * LANES + lane
            cb = ((ix * GRID + iy) & 3) * 4
            z00 = plsc.load_gather(vq[p], [row, cb])
            z01 = plsc.load_gather(vq[p], [row, cb + 1])
            z10 = plsc.load_gather(vq[p], [row, cb + 2])
            z11 = plsc.load_gather(vq[p], [row, cb + 3])
            a = z00 + (z01 - z00) * wy
            b = z10 + (z11 - z10) * wy
            ob[p][sl] = a + (b - a) * wx
            return carry

        lax.fori_loop(0, VECS, body, 0)
        pltpu.sync_copy(ob[p], out_hbm.at[pl.ds(off, CHUNK)])

    load_xy(0, base)
    compute_idx(0)
    cp = fire(0)
    for ci in range(NCHUNK):
        p = ci & 1
        q = p ^ 1
        nxt = None
        if ci + 1 < NCHUNK:
            load_xy(q, base + (ci + 1) * CHUNK)
            compute_idx(q)
            nxt = fire(q)
        cp.wait()
        mix(p, base + ci * CHUNK)
        cp = nxt


_interp = functools.partial(
    pl.kernel,
    out_type=jax.ShapeDtypeStruct((NPAD,), jnp.float32),
    mesh=plsc.VectorSubcoreMesh(core_axis_name="c", subcore_axis_name="s"),
    compiler_params=pltpu.CompilerParams(
        needs_layout_passes=False, use_tc_tiling_on_sc=False),
    scratch_types=[
        pltpu.VMEM((CHUNK,), jnp.float32),
        pltpu.VMEM((CHUNK,), jnp.float32),
        pltpu.VMEM((CHUNK,), jnp.float32),
        pltpu.VMEM((CHUNK,), jnp.float32),
        pltpu.VMEM((CHUNK,), jnp.int32),
        pltpu.VMEM((CHUNK,), jnp.int32),
        pltpu.VMEM((CHUNK, 16), jnp.float32),
        pltpu.VMEM((CHUNK, 16), jnp.float32),
        pltpu.VMEM((CHUNK,), jnp.float32),
        pltpu.VMEM((CHUNK,), jnp.float32),
        pltpu.SemaphoreType.DMA,
        pltpu.SemaphoreType.DMA,
    ],
)(_interp_body)


def kernel(points_to_interpolate, xs, ys, zs, repeats=1):
    n = points_to_interpolate.shape[0]
    xp = jnp.pad(points_to_interpolate[:, 0], (0, NPAD - n))
    yp = jnp.pad(points_to_interpolate[:, 1], (0, NPAD - n))
    table = _build(zs.reshape(-1))
    out = _interp(xp, yp, table)
    return out[:n]


# speedup vs baseline: 42.2373x; 1.2733x over previous
import functools

import jax
import jax.numpy as jnp
from jax import lax
from jax.experimental import pallas as pl
from jax.experimental.pallas import tpu as pltpu
from jax.experimental.pallas import tpu_sc as plsc

NPAD = 1_048_576
NW = 32
PER_W = NPAD // NW
CHUNK = 2048
NCHUNK = PER_W // CHUNK
LANES = 16
VECS = CHUNK // LANES
GRID = 4096
NCELL = (GRID - 2) * GRID + GRID - 2 + 2
ROWS16 = NCELL // 4

BCELL = NCELL // NW
BCH = 8192
BSIZES = [BCH] * 63 + [BCELL - 63 * BCH]


def _build_body(zf_hbm, tab_hbm, slab1a, slab1b, slab2a, slab2b,
                ob0, ob1, sem0, sem1, lsem0, lsem1):
    slab1 = (slab1a, slab1b)
    slab2 = (slab2a, slab2b)
    ob = (ob0, ob1)
    sem = (sem0, sem1)
    lsem = (lsem0, lsem1)
    c = lax.axis_index("c")
    s = lax.axis_index("s")
    base = (s * 2 + c) * BCELL
    is_last_w = (s * 2 + c) == NW - 1

    lane = lax.iota(jnp.int32, LANES)
    rc = lax.shift_right_logical(lane, 2)
    cv = (lane & 3) * 4

    offs = []
    off = 0
    for cb in BSIZES:
        offs.append(off)
        off += cb

    def prefetch(ci):
        cb = BSIZES[ci]
        c0 = base + offs[ci]
        p = ci & 1
        last = ci == len(BSIZES) - 1
        b2 = 16 if last else 0
        cp1 = pltpu.async_copy(zf_hbm.at[pl.ds(c0, cb + 16)],
                               slab1[p].at[pl.ds(0, cb + 16)], lsem[p])
        if last:
            @pl.when(is_last_w)
            def _():
                pltpu.async_copy(zf_hbm.at[pl.ds(c0 + GRID - b2, cb + 16)],
                                 slab2[p].at[pl.ds(0, cb + 16)], lsem[p])

            @pl.when(jnp.logical_not(is_last_w))
            def _():
                pltpu.async_copy(zf_hbm.at[pl.ds(c0 + GRID - b2, cb + 24)],
                                 slab2[p].at[pl.ds(0, cb + 24)], lsem[p])
            cp2 = pltpu.make_async_copy(
                zf_hbm.at[pl.ds(c0 + GRID - b2, cb + 16)],
                slab2[p].at[pl.ds(0, cb + 16)], lsem[p])
            cp2b = pltpu.make_async_copy(
                zf_hbm.at[pl.ds(c0 + GRID - b2, cb + 24)],
                slab2[p].at[pl.ds(0, cb + 24)], lsem[p])
            return [cp1, (cp2, cp2b)]
        cp2 = pltpu.async_copy(zf_hbm.at[pl.ds(c0 + GRID, cb + 16)],
                               slab2[p].at[pl.ds(0, cb + 16)], lsem[p])
        return [cp1, cp2]

    def wait_loads(cps, ci):
        last = ci == len(BSIZES) - 1
        cps[0].wait()
        if last:
            cp2, cp2b = cps[1]

            @pl.when(is_last_w)
            def _():
                cp2.wait()

            @pl.when(jnp.logical_not(is_last_w))
            def _():
                cp2b.wait()
        else:
            cps[1].wait()

    ocps = [None, None]
    lcps = prefetch(0)
    for ci, cb in enumerate(BSIZES):
        c0 = base + offs[ci]
        p = ci & 1
        last = ci == len(BSIZES) - 1
        b2 = 16 if last else 0
        nxt = prefetch(ci + 1) if ci + 1 < len(BSIZES) else None
        wait_loads(lcps, ci)
        if ocps[p] is not None:
            ocps[p].wait()

        def it(i, carry, p=p, b2=b2):
            for g in range(4):
                u0 = i * 64 + g * 16
                z00 = slab1[p][pl.ds(u0, LANES)]
                z01 = slab1[p][pl.ds(u0 + 1, LANES)]
                z10 = slab2[p][pl.ds(u0 + b2, LANES)]
                z11 = slab2[p][pl.ds(u0 + b2 + 1, LANES)]
                rows = i * 16 + (g * 4) + rc
                plsc.store_scatter(ob[p], [rows, cv], z00)
                plsc.store_scatter(ob[p], [rows, cv + 1], z01)
                plsc.store_scatter(ob[p], [rows, cv + 2], z10)
                plsc.store_scatter(ob[p], [rows, cv + 3], z11)
            return carry

        lax.fori_loop(0, cb // 64, it, 0)
        ocps[p] = pltpu.async_copy(
            ob[p].at[pl.ds(0, cb // 4)], tab_hbm.at[pl.ds(c0 // 4, cb // 4)],
            sem[p])
        lcps = nxt
    for cp in ocps:
        if cp is not None:
            cp.wait()


_build = functools.partial(
    pl.kernel,
    out_type=jax.ShapeDtypeStruct((ROWS16, 16), jnp.float32),
    mesh=plsc.VectorSubcoreMesh(core_axis_name="c", subcore_axis_name="s"),
    compiler_params=pltpu.CompilerParams(
        needs_layout_passes=False, use_tc_tiling_on_sc=False),
    scratch_types=[
        pltpu.VMEM((BCH + 16,), jnp.float32),
        pltpu.VMEM((BCH + 16,), jnp.float32),
        pltpu.VMEM((BCH + 24,), jnp.float32),
        pltpu.VMEM((BCH + 24,), jnp.float32),
        pltpu.VMEM((BCH // 4, 16), jnp.float32),
        pltpu.VMEM((BCH // 4, 16), jnp.float32),
        pltpu.SemaphoreType.DMA,
        pltpu.SemaphoreType.DMA,
        pltpu.SemaphoreType.DMA,
        pltpu.SemaphoreType.DMA,
    ],
)(_build_body)


def _interp_body(x_hbm, y_hbm, zq_hbm, out_hbm,
                 xb0, xb1, yb0, yb1, ib0, ib1, vq0, vq1, ob0, ob1,
                 sem0, sem1):
    xb = (xb0, xb1)
    yb = (yb0, yb1)
    ib = (ib0, ib1)
    vq = (vq0, vq1)
    ob = (ob0, ob1)
    sem = (sem0, sem1)

    c = lax.axis_index("c")
    s = lax.axis_index("s")
    base = (s * 2 + c) * PER_W

    lane = lax.iota(jnp.int32, LANES)

    def load_xy(p, off):
        pltpu.sync_copy(x_hbm.at[pl.ds(off, CHUNK)], xb[p])
        pltpu.sync_copy(y_hbm.at[pl.ds(off, CHUNK)], yb[p])

    def cell_xy(xv, yv):
        ix = jnp.clip(xv.astype(jnp.int32), 0, GRID - 2)
        iy = jnp.clip(yv.astype(jnp.int32), 0, GRID - 2)
        return ix, iy

    def compute_idx(p):
        def body(i, carry):
            sl = pl.ds(i * LANES, LANES)
            ix, iy = cell_xy(xb[p][sl], yb[p][sl])
            cell = ix * GRID + iy
            ib[p][sl] = lax.shift_right_logical(cell, 2)
            return carry

        lax.fori_loop(0, VECS, body, 0)

    def fire(p):
        return pltpu.async_copy(zq_hbm.at[ib[p]], vq[p], sem[p])

    def mix(p, off):
        def body(i, carry):
            sl = pl.ds(i * LANES, LANES)
            xv = xb[p][sl]
            yv = yb[p][sl]
            ix, iy = cell_xy(xv, yv)
            wx = xv - ix.astype(jnp.float32)
            wy = yv - iy.astype(jnp.float32)
            row = i * LANES + lane
            cb = ((ix * GRID + iy) & 3) * 4
            z00 = plsc.load_gather(vq[p], [row, cb])
            z01 = plsc.load_gather(vq[p], [row, cb + 1])
            z10 = plsc.load_gather(vq[p], [row, cb + 2])
            z11 = plsc.load_gather(vq[p], [row, cb + 3])
            a = z00 + (z01 - z00) * wy
            b = z10 + (z11 - z10) * wy
            ob[p][sl] = a + (b - a) * wx
            return carry

        lax.fori_loop(0, VECS, body, 0)
        pltpu.sync_copy(ob[p], out_hbm.at[pl.ds(off, CHUNK)])

    load_xy(0, base)
    compute_idx(0)
    cp = fire(0)
    for ci in range(NCHUNK):
        p = ci & 1
        q = p ^ 1
        nxt = None
        if ci + 1 < NCHUNK:
            load_xy(q, base + (ci + 1) * CHUNK)
            compute_idx(q)
            nxt = fire(q)
        cp.wait()
        mix(p, base + ci * CHUNK)
        cp = nxt


_interp = functools.partial(
    pl.kernel,
    out_type=jax.ShapeDtypeStruct((NPAD,), jnp.float32),
    mesh=plsc.VectorSubcoreMesh(core_axis_name="c", subcore_axis_name="s"),
    compiler_params=pltpu.CompilerParams(
        needs_layout_passes=False, use_tc_tiling_on_sc=False),
    scratch_types=[
        pltpu.VMEM((CHUNK,), jnp.float32),
        pltpu.VMEM((CHUNK,), jnp.float32),
        pltpu.VMEM((CHUNK,), jnp.float32),
        pltpu.VMEM((CHUNK,), jnp.float32),
        pltpu.VMEM((CHUNK,), jnp.int32),
        pltpu.VMEM((CHUNK,), jnp.int32),
        pltpu.VMEM((CHUNK, 16), jnp.float32),
        pltpu.VMEM((CHUNK, 16), jnp.float32),
        pltpu.VMEM((CHUNK,), jnp.float32),
        pltpu.VMEM((CHUNK,), jnp.float32),
        pltpu.SemaphoreType.DMA,
        pltpu.SemaphoreType.DMA,
    ],
)(_interp_body)


def kernel(points_to_interpolate, xs, ys, zs, repeats=1):
    n = points_to_interpolate.shape[0]
    xp = jnp.pad(points_to_interpolate[:, 0], (0, NPAD - n))
    yp = jnp.pad(points_to_interpolate[:, 1], (0, NPAD - n))
    table = _build(zs.reshape(-1))
    out = _interp(xp, yp, table)
    return out[:n]


# speedup vs baseline: 69.2510x; 1.6396x over previous
import functools

import jax
import jax.numpy as jnp
from jax import lax
from jax.experimental import pallas as pl
from jax.experimental.pallas import tpu as pltpu
from jax.experimental.pallas import tpu_sc as plsc

NPTS = 1_000_000
NW = 32
PER_W = 31_248
TAIL = NPTS - NW * PER_W
CHUNK = 2048
PCHUNKS = [CHUNK] * 15 + [PER_W - 15 * CHUNK]
LANES = 16
GRID = 4096
NCELL = (GRID - 2) * GRID + GRID - 2 + 2
ROWS16 = NCELL // 4

BCELL = NCELL // NW
BCH = 8192
BSIZES = [BCH] * 63 + [BCELL - 63 * BCH]


def _build_body(zf_hbm, tab_hbm, slab1a, slab1b, slab2a, slab2b,
                ob0, ob1, sem0, sem1, lsem0, lsem1):
    slab1 = (slab1a, slab1b)
    slab2 = (slab2a, slab2b)
    ob = (ob0, ob1)
    sem = (sem0, sem1)
    lsem = (lsem0, lsem1)
    c = lax.axis_index("c")
    s = lax.axis_index("s")
    base = (s * 2 + c) * BCELL
    is_last_w = (s * 2 + c) == NW - 1

    lane = lax.iota(jnp.int32, LANES)
    rc = lax.shift_right_logical(lane, 2)
    cv = (lane & 3) * 4

    offs = []
    off = 0
    for cb in BSIZES:
        offs.append(off)
        off += cb

    def prefetch(ci):
        cb = BSIZES[ci]
        c0 = base + offs[ci]
        p = ci & 1
        last = ci == len(BSIZES) - 1
        b2 = 16 if last else 0
        cp1 = pltpu.async_copy(zf_hbm.at[pl.ds(c0, cb + 16)],
                               slab1[p].at[pl.ds(0, cb + 16)], lsem[p])
        if last:
            @pl.when(is_last_w)
            def _():
                pltpu.async_copy(zf_hbm.at[pl.ds(c0 + GRID - b2, cb + 16)],
                                 slab2[p].at[pl.ds(0, cb + 16)], lsem[p])

            @pl.when(jnp.logical_not(is_last_w))
            def _():
                pltpu.async_copy(zf_hbm.at[pl.ds(c0 + GRID - b2, cb + 24)],
                                 slab2[p].at[pl.ds(0, cb + 24)], lsem[p])
            cp2 = pltpu.make_async_copy(
                zf_hbm.at[pl.ds(c0 + GRID - b2, cb + 16)],
                slab2[p].at[pl.ds(0, cb + 16)], lsem[p])
            cp2b = pltpu.make_async_copy(
                zf_hbm.at[pl.ds(c0 + GRID - b2, cb + 24)],
                slab2[p].at[pl.ds(0, cb + 24)], lsem[p])
            return [cp1, (cp2, cp2b)]
        cp2 = pltpu.async_copy(zf_hbm.at[pl.ds(c0 + GRID, cb + 16)],
                               slab2[p].at[pl.ds(0, cb + 16)], lsem[p])
        return [cp1, cp2]

    def wait_loads(cps, ci):
        last = ci == len(BSIZES) - 1
        cps[0].wait()
        if last:
            cp2, cp2b = cps[1]

            @pl.when(is_last_w)
            def _():
                cp2.wait()

            @pl.when(jnp.logical_not(is_last_w))
            def _():
                cp2b.wait()
        else:
            cps[1].wait()

    ocps = [None, None]
    lcps = prefetch(0)
    for ci, cb in enumerate(BSIZES):
        c0 = base + offs[ci]
        p = ci & 1
        last = ci == len(BSIZES) - 1
        b2 = 16 if last else 0
        nxt = prefetch(ci + 1) if ci + 1 < len(BSIZES) else None
        wait_loads(lcps, ci)
        if ocps[p] is not None:
            ocps[p].wait()

        def it(i, carry, p=p, b2=b2):
            for g in range(4):
                u0 = i * 64 + g * 16
                z00 = slab1[p][pl.ds(u0, LANES)]
                z01 = slab1[p][pl.ds(u0 + 1, LANES)]
                z10 = slab2[p][pl.ds(u0 + b2, LANES)]
                z11 = slab2[p][pl.ds(u0 + b2 + 1, LANES)]
                rows = i * 16 + (g * 4) + rc
                plsc.store_scatter(ob[p], [rows, cv], z00)
                plsc.store_scatter(ob[p], [rows, cv + 1], z01)
                plsc.store_scatter(ob[p], [rows, cv + 2], z10)
                plsc.store_scatter(ob[p], [rows, cv + 3], z11)
            return carry

        lax.fori_loop(0, cb // 64, it, 0)
        ocps[p] = pltpu.async_copy(
            ob[p].at[pl.ds(0, cb // 4)], tab_hbm.at[pl.ds(c0 // 4, cb // 4)],
            sem[p])
        lcps = nxt
    for cp in ocps:
        if cp is not None:
            cp.wait()


_build = functools.partial(
    pl.kernel,
    out_type=jax.ShapeDtypeStruct((ROWS16, 16), jnp.float32),
    mesh=plsc.VectorSubcoreMesh(core_axis_name="c", subcore_axis_name="s"),
    compiler_params=pltpu.CompilerParams(
        needs_layout_passes=False, use_tc_tiling_on_sc=False),
    scratch_types=[
        pltpu.VMEM((BCH + 16,), jnp.float32),
        pltpu.VMEM((BCH + 16,), jnp.float32),
        pltpu.VMEM((BCH + 24,), jnp.float32),
        pltpu.VMEM((BCH + 24,), jnp.float32),
        pltpu.VMEM((BCH // 4, 16), jnp.float32),
        pltpu.VMEM((BCH // 4, 16), jnp.float32),
        pltpu.SemaphoreType.DMA,
        pltpu.SemaphoreType.DMA,
        pltpu.SemaphoreType.DMA,
        pltpu.SemaphoreType.DMA,
    ],
)(_build_body)


def _interp_body(x_hbm, y_hbm, zq_hbm, out_hbm,
                 xb0, xb1, yb0, yb1, ib0, ib1, vq0, vq1, ob0, ob1,
                 sem0, sem1):
    xb = (xb0, xb1)
    yb = (yb0, yb1)
    ib = (ib0, ib1)
    vq = (vq0, vq1)
    ob = (ob0, ob1)
    sem = (sem0, sem1)

    c = lax.axis_index("c")
    s = lax.axis_index("s")
    wid = s * 2 + c
    base = wid * PER_W

    lane = lax.iota(jnp.int32, LANES)

    def load_xy(p, off, sz):
        pltpu.sync_copy(x_hbm.at[pl.ds(off, sz)], xb[p].at[pl.ds(0, sz)])
        pltpu.sync_copy(y_hbm.at[pl.ds(off, sz)], yb[p].at[pl.ds(0, sz)])

    def cell_xy(xv, yv):
        ix = jnp.clip(xv.astype(jnp.int32), 0, GRID - 2)
        iy = jnp.clip(yv.astype(jnp.int32), 0, GRID - 2)
        return ix, iy

    def compute_idx(p, sz):
        def body(i, carry):
            sl = pl.ds(i * LANES, LANES)
            ix, iy = cell_xy(xb[p][sl], yb[p][sl])
            cell = ix * GRID + iy
            ib[p][sl] = lax.shift_right_logical(cell, 2)
            return carry

        lax.fori_loop(0, sz // LANES, body, 0)

    def fire(p, sz):
        if sz == CHUNK:
            return pltpu.async_copy(zq_hbm.at[ib[p]], vq[p], sem[p])
        return pltpu.async_copy(zq_hbm.at[ib[p].at[pl.ds(0, sz)]],
                                vq[p].at[pl.ds(0, sz)], sem[p])

    def mix(p, off, sz):
        def body(i, carry):
            sl = pl.ds(i * LANES, LANES)
            xv = xb[p][sl]
            yv = yb[p][sl]
            ix, iy = cell_xy(xv, yv)
            wx = xv - ix.astype(jnp.float32)
            wy = yv - iy.astype(jnp.float32)
            row = i * LANES + lane
            cb = ((ix * GRID + iy) & 3) * 4
            z00 = plsc.load_gather(vq[p], [row, cb])
            z01 = plsc.load_gather(vq[p], [row, cb + 1])
            z10 = plsc.load_gather(vq[p], [row, cb + 2])
            z11 = plsc.load_gather(vq[p], [row, cb + 3])
            a = z00 + (z01 - z00) * wy
            b = z10 + (z11 - z10) * wy
            ob[p][sl] = a + (b - a) * wx
            return carry

        lax.fori_loop(0, sz // LANES, body, 0)
        pltpu.sync_copy(ob[p].at[pl.ds(0, sz)], out_hbm.at[pl.ds(off, sz)])

    offs = []
    o = 0
    for sz in PCHUNKS:
        offs.append(o)
        o += sz

    load_xy(0, base, PCHUNKS[0])
    compute_idx(0, PCHUNKS[0])
    cp = fire(0, PCHUNKS[0])
    for ci, sz in enumerate(PCHUNKS):
        p = ci & 1
        q = p ^ 1
        nxt = None
        if ci + 1 < len(PCHUNKS):
            nsz = PCHUNKS[ci + 1]
            load_xy(q, base + offs[ci + 1], nsz)
            compute_idx(q, nsz)
            nxt = fire(q, nsz)
        cp.wait()
        mix(p, base + offs[ci], sz)
        cp = nxt

    @pl.when(wid == NW - 1)
    def _():
        t0 = NW * PER_W
        load_xy(0, t0, TAIL)
        compute_idx(0, TAIL)
        fire(0, TAIL).wait()
        mix(0, t0, TAIL)


_interp = functools.partial(
    pl.kernel,
    out_type=jax.ShapeDtypeStruct((NPTS,), jnp.float32),
    mesh=plsc.VectorSubcoreMesh(core_axis_name="c", subcore_axis_name="s"),
    compiler_params=pltpu.CompilerParams(
        needs_layout_passes=False, use_tc_tiling_on_sc=False),
    scratch_types=[
        pltpu.VMEM((CHUNK,), jnp.float32),
        pltpu.VMEM((CHUNK,), jnp.float32),
        pltpu.VMEM((CHUNK,), jnp.float32),
        pltpu.VMEM((CHUNK,), jnp.float32),
        pltpu.VMEM((CHUNK,), jnp.int32),
        pltpu.VMEM((CHUNK,), jnp.int32),
        pltpu.VMEM((CHUNK, 16), jnp.float32),
        pltpu.VMEM((CHUNK, 16), jnp.float32),
        pltpu.VMEM((CHUNK,), jnp.float32),
        pltpu.VMEM((CHUNK,), jnp.float32),
        pltpu.SemaphoreType.DMA,
        pltpu.SemaphoreType.DMA,
    ],
)(_interp_body)


def kernel(points_to_interpolate, xs, ys, zs, repeats=1):
    x = points_to_interpolate[:, 0]
    y = points_to_interpolate[:, 1]
    table = _build(zs.reshape(-1))
    return _interp(x, y, table)


# speedup vs baseline: 69.8640x; 1.0089x over previous
import functools

import jax
import jax.numpy as jnp
from jax import lax
from jax.experimental import pallas as pl
from jax.experimental.pallas import tpu as pltpu
from jax.experimental.pallas import tpu_sc as plsc

NPTS = 1_000_000
NW = 32
PER_W = 31_248
TAIL = NPTS - NW * PER_W
CHUNK = 2048
PCHUNKS = [CHUNK] * 15 + [PER_W - 15 * CHUNK]
LANES = 16
GRID = 4096
NCELL = (GRID - 2) * GRID + GRID - 2 + 2
ROWS16 = NCELL // 4

BCELL = NCELL // NW
BCH = 10240
BSIZES = [BCH] * 51 + [BCELL - 51 * BCH]


def _build_body(zf_hbm, tab_hbm, slab1a, slab1b, slab2a, slab2b,
                ob0, ob1, sem0, sem1, lsem0, lsem1):
    slab1 = (slab1a, slab1b)
    slab2 = (slab2a, slab2b)
    ob = (ob0, ob1)
    sem = (sem0, sem1)
    lsem = (lsem0, lsem1)
    c = lax.axis_index("c")
    s = lax.axis_index("s")
    base = (s * 2 + c) * BCELL
    is_last_w = (s * 2 + c) == NW - 1

    lane = lax.iota(jnp.int32, LANES)
    rc = lax.shift_right_logical(lane, 2)
    cv = (lane & 3) * 4

    offs = []
    off = 0
    for cb in BSIZES:
        offs.append(off)
        off += cb

    def prefetch(ci):
        cb = BSIZES[ci]
        c0 = base + offs[ci]
        p = ci & 1
        last = ci == len(BSIZES) - 1
        b2 = 16 if last else 0
        cp1 = pltpu.async_copy(zf_hbm.at[pl.ds(c0, cb + 16)],
                               slab1[p].at[pl.ds(0, cb + 16)], lsem[p])
        if last:
            @pl.when(is_last_w)
            def _():
                pltpu.async_copy(zf_hbm.at[pl.ds(c0 + GRID - b2, cb + 16)],
                                 slab2[p].at[pl.ds(0, cb + 16)], lsem[p])

            @pl.when(jnp.logical_not(is_last_w))
            def _():
                pltpu.async_copy(zf_hbm.at[pl.ds(c0 + GRID - b2, cb + 24)],
                                 slab2[p].at[pl.ds(0, cb + 24)], lsem[p])
            cp2 = pltpu.make_async_copy(
                zf_hbm.at[pl.ds(c0 + GRID - b2, cb + 16)],
                slab2[p].at[pl.ds(0, cb + 16)], lsem[p])
            cp2b = pltpu.make_async_copy(
                zf_hbm.at[pl.ds(c0 + GRID - b2, cb + 24)],
                slab2[p].at[pl.ds(0, cb + 24)], lsem[p])
            return [cp1, (cp2, cp2b)]
        cp2 = pltpu.async_copy(zf_hbm.at[pl.ds(c0 + GRID, cb + 16)],
                               slab2[p].at[pl.ds(0, cb + 16)], lsem[p])
        return [cp1, cp2]

    def wait_loads(cps, ci):
        last = ci == len(BSIZES) - 1
        cps[0].wait()
        if last:
            cp2, cp2b = cps[1]

            @pl.when(is_last_w)
            def _():
                cp2.wait()

            @pl.when(jnp.logical_not(is_last_w))
            def _():
                cp2b.wait()
        else:
            cps[1].wait()

    ocps = [None, None]
    lcps = prefetch(0)
    for ci, cb in enumerate(BSIZES):
        c0 = base + offs[ci]
        p = ci & 1
        last = ci == len(BSIZES) - 1
        b2 = 16 if last else 0
        nxt = prefetch(ci + 1) if ci + 1 < len(BSIZES) else None
        wait_loads(lcps, ci)
        if ocps[p] is not None:
            ocps[p].wait()

        def it(i, carry, p=p, b2=b2):
            for g in range(4):
                u0 = i * 64 + g * 16
                z00 = slab1[p][pl.ds(u0, LANES)]
                z01 = slab1[p][pl.ds(u0 + 1, LANES)]
                z10 = slab2[p][pl.ds(u0 + b2, LANES)]
                z11 = slab2[p][pl.ds(u0 + b2 + 1, LANES)]
                rows = i * 16 + (g * 4) + rc
                plsc.store_scatter(ob[p], [rows, cv], z00)
                plsc.store_scatter(ob[p], [rows, cv + 1], z01)
                plsc.store_scatter(ob[p], [rows, cv + 2], z10)
                plsc.store_scatter(ob[p], [rows, cv + 3], z11)
            return carry

        lax.fori_loop(0, cb // 64, it, 0)
        ocps[p] = pltpu.async_copy(
            ob[p].at[pl.ds(0, cb // 4)], tab_hbm.at[pl.ds(c0 // 4, cb // 4)],
            sem[p])
        lcps = nxt
    for cp in ocps:
        if cp is not None:
            cp.wait()


_build = functools.partial(
    pl.kernel,
    out_type=jax.ShapeDtypeStruct((ROWS16, 16), jnp.float32),
    mesh=plsc.VectorSubcoreMesh(core_axis_name="c", subcore_axis_name="s"),
    compiler_params=pltpu.CompilerParams(
        needs_layout_passes=False, use_tc_tiling_on_sc=False),
    scratch_types=[
        pltpu.VMEM((BCH + 16,), jnp.float32),
        pltpu.VMEM((BCH + 16,), jnp.float32),
        pltpu.VMEM((BCH + 24,), jnp.float32),
        pltpu.VMEM((BCH + 24,), jnp.float32),
        pltpu.VMEM((BCH // 4, 16), jnp.float32),
        pltpu.VMEM((BCH // 4, 16), jnp.float32),
        pltpu.SemaphoreType.DMA,
        pltpu.SemaphoreType.DMA,
        pltpu.SemaphoreType.DMA,
        pltpu.SemaphoreType.DMA,
    ],
)(_build_body)


def _interp_body(x_hbm, y_hbm, zq_hbm, out_hbm,
                 xb0, xb1, yb0, yb1, ib0, ib1, vq0, vq1, ob0, ob1,
                 sem0, sem1):
    xb = (xb0, xb1)
    yb = (yb0, yb1)
    ib = (ib0, ib1)
    vq = (vq0, vq1)
    ob = (ob0, ob1)
    sem = (sem0, sem1)

    c = lax.axis_index("c")
    s = lax.axis_index("s")
    wid = s * 2 + c
    base = wid * PER_W

    lane = lax.iota(jnp.int32, LANES)

    def load_xy(p, off, sz):
        pltpu.sync_copy(x_hbm.at[pl.ds(off, sz)], xb[p].at[pl.ds(0, sz)])
        pltpu.sync_copy(y_hbm.at[pl.ds(off, sz)], yb[p].at[pl.ds(0, sz)])

    def cell_xy(xv, yv):
        ix = jnp.clip(xv.astype(jnp.int32), 0, GRID - 2)
        iy = jnp.clip(yv.astype(jnp.int32), 0, GRID - 2)
        return ix, iy

    def compute_idx(p, sz):
        def body(i, carry):
            sl = pl.ds(i * LANES, LANES)
            ix, iy = cell_xy(xb[p][sl], yb[p][sl])
            cell = ix * GRID + iy
            ib[p][sl] = lax.shift_right_logical(cell, 2)
            return carry

        lax.fori_loop(0, sz // LANES, body, 0)

    def fire(p, sz):
        if sz == CHUNK:
            return pltpu.async_copy(zq_hbm.at[ib[p]], vq[p], sem[p])
        return pltpu.async_copy(zq_hbm.at[ib[p].at[pl.ds(0, sz)]],
                                vq[p].at[pl.ds(0, sz)], sem[p])

    def mix(p, off, sz):
        def body(i, carry):
            sl = pl.ds(i * LANES, LANES)
            xv = xb[p][sl]
            yv = yb[p][sl]
            ix, iy = cell_xy(xv, yv)
            wx = xv - ix.astype(jnp.float32)
            wy = yv - iy.astype(jnp.float32)
            row = i * LANES + lane
            cb = ((ix * GRID + iy) & 3) * 4
            z00 = plsc.load_gather(vq[p], [row, cb])
            z01 = plsc.load_gather(vq[p], [row, cb + 1])
            z10 = plsc.load_gather(vq[p], [row, cb + 2])
            z11 = plsc.load_gather(vq[p], [row, cb + 3])
            a = z00 + (z01 - z00) * wy
            b = z10 + (z11 - z10) * wy
            ob[p][sl] = a + (b - a) * wx
            return carry

        lax.fori_loop(0, sz // LANES, body, 0)
        pltpu.sync_copy(ob[p].at[pl.ds(0, sz)], out_hbm.at[pl.ds(off, sz)])

    offs = []
    o = 0
    for sz in PCHUNKS:
        offs.append(o)
        o += sz

    load_xy(0, base, PCHUNKS[0])
    compute_idx(0, PCHUNKS[0])
    cp = fire(0, PCHUNKS[0])
    for ci, sz in enumerate(PCHUNKS):
        p = ci & 1
        q = p ^ 1
        nxt = None
        if ci + 1 < len(PCHUNKS):
            nsz = PCHUNKS[ci + 1]
            load_xy(q, base + offs[ci + 1], nsz)
            compute_idx(q, nsz)
            nxt = fire(q, nsz)
        cp.wait()
        mix(p, base + offs[ci], sz)
        cp = nxt

    @pl.when(wid == NW - 1)
    def _():
        t0 = NW * PER_W
        load_xy(0, t0, TAIL)
        compute_idx(0, TAIL)
        fire(0, TAIL).wait()
        mix(0, t0, TAIL)


_interp = functools.partial(
    pl.kernel,
    out_type=jax.ShapeDtypeStruct((NPTS,), jnp.float32),
    mesh=plsc.VectorSubcoreMesh(core_axis_name="c", subcore_axis_name="s"),
    compiler_params=pltpu.CompilerParams(
        needs_layout_passes=False, use_tc_tiling_on_sc=False),
    scratch_types=[
        pltpu.VMEM((CHUNK,), jnp.float32),
        pltpu.VMEM((CHUNK,), jnp.float32),
        pltpu.VMEM((CHUNK,), jnp.float32),
        pltpu.VMEM((CHUNK,), jnp.float32),
        pltpu.VMEM((CHUNK,), jnp.int32),
        pltpu.VMEM((CHUNK,), jnp.int32),
        pltpu.VMEM((CHUNK, 16), jnp.float32),
        pltpu.VMEM((CHUNK, 16), jnp.float32),
        pltpu.VMEM((CHUNK,), jnp.float32),
        pltpu.VMEM((CHUNK,), jnp.float32),
        pltpu.SemaphoreType.DMA,
        pltpu.SemaphoreType.DMA,
    ],
)(_interp_body)


def kernel(points_to_interpolate, xs, ys, zs, repeats=1):
    x = points_to_interpolate[:, 0]
    y = points_to_interpolate[:, 1]
    table = _build(zs.reshape(-1))
    return _interp(x, y, table)
